# baseline (speedup 1.0000x reference)
import functools

import jax
import jax.numpy as jnp
from jax.experimental import pallas as pl
from jax.experimental.pallas import tpu as pltpu


ROWS_PER_BLOCK = 32


def _rpe_block(x_ref, tbl_ref, out_ref):
    seq_len = x_ref.shape[1]
    i0 = pl.program_id(0) * ROWS_PER_BLOCK
    xv = x_ref[0]
    for r in range(ROWS_PER_BLOCK):
        k = r % 8
        base = pl.multiple_of(seq_len - i0 - (r - k), 8)
        out_ref[r] = xv + tbl_ref[k, pl.ds(base, seq_len), :]


def kernel(x, embed_table):
    batch, seq_len, d_model = x.shape
    n_rows = embed_table.shape[0]
    padded = ((n_rows + 7 + 7) // 8) * 8
    shifted = jnp.stack(
        [
            jnp.pad(embed_table, ((k, padded - n_rows - k), (0, 0)))
            for k in range(8)
        ]
    )
    grid = (seq_len // ROWS_PER_BLOCK,)
    out = pl.pallas_call(
        _rpe_block,
        grid=grid,
        in_specs=[
            pl.BlockSpec((batch, seq_len, d_model), lambda i: (0, 0, 0)),
            pl.BlockSpec(shifted.shape, lambda i: (0, 0, 0)),
        ],
        out_specs=pl.BlockSpec(
            (ROWS_PER_BLOCK, seq_len, d_model), lambda i: (i, 0, 0)
        ),
        out_shape=jax.ShapeDtypeStruct((seq_len, seq_len, d_model), x.dtype),
        compiler_params=pltpu.CompilerParams(
            dimension_semantics=("parallel",)
        ),
    )(x, shifted)
    return out


# speedup vs baseline: 1.2092x; 1.2092x over previous
import functools

import jax
import jax.numpy as jnp
from jax.experimental import pallas as pl
from jax.experimental.pallas import tpu as pltpu


ROWS_PER_BLOCK = 16


def _rpe_block(x_ref, tbl_ref, out_ref):
    seq_len = x_ref.shape[1]
    i0 = pl.program_id(0) * ROWS_PER_BLOCK
    xv = x_ref[0]
    base = pl.multiple_of(seq_len - i0 - ROWS_PER_BLOCK, 8)
    sup = tbl_ref[pl.ds(base, seq_len + ROWS_PER_BLOCK), :]
    for r in range(ROWS_PER_BLOCK):
        off = ROWS_PER_BLOCK - r
        out_ref[r] = xv + sup[off : off + seq_len]


def kernel(x, embed_table):
    batch, seq_len, d_model = x.shape
    grid = (seq_len // ROWS_PER_BLOCK,)
    out = pl.pallas_call(
        _rpe_block,
        grid=grid,
        in_specs=[
            pl.BlockSpec((batch, seq_len, d_model), lambda i: (0, 0, 0)),
            pl.BlockSpec(embed_table.shape, lambda i: (0, 0)),
        ],
        out_specs=pl.BlockSpec(
            (ROWS_PER_BLOCK, seq_len, d_model), lambda i: (i, 0, 0)
        ),
        out_shape=jax.ShapeDtypeStruct((seq_len, seq_len, d_model), x.dtype),
        compiler_params=pltpu.CompilerParams(
            dimension_semantics=("parallel",)
        ),
    )(x, embed_table)
    return out
